# Initial kernel scaffold; baseline (speedup 1.0000x reference)
import jax
import jax.numpy as jnp
from jax.experimental import pallas as pl


def kernel(x, batch, W1, b1, W2, b2):
    raise NotImplementedError("write your pallas kernel here")


# speedup vs baseline: 3.0365x; 3.0365x over previous
import functools

import jax
import jax.numpy as jnp
from jax import lax
from jax.experimental import pallas as pl
from jax.experimental.pallas import tpu as pltpu
from jax.experimental.pallas import tpu_sc as plsc

N = 320000
D_IN = 128
D_HID = 256
S = 10000

RB = 1280
NC = 2
NS = 16
NW = NC * NS
RPW = N // NW
K = 80
NCHUNK = RPW // K
SEG_PER_TILE = 624
SEG_TAIL = S - NS * SEG_PER_TILE


def _z_body(x_ref, w1t_ref, b1_ref, w2t_ref, z_ref):
    h = jnp.dot(x_ref[...], w1t_ref[...], preferred_element_type=jnp.float32)
    phi = jnp.tanh(h + b1_ref[...])
    z_ref[...] = jnp.dot(phi, w2t_ref[...], preferred_element_type=jnp.float32)


def _compute_z(x, W1T, b1, W2T):
    return pl.pallas_call(
        _z_body,
        grid=(N // RB,),
        in_specs=[
            pl.BlockSpec((RB, D_IN), lambda i: (i, 0)),
            pl.BlockSpec((D_IN, D_HID), lambda i: (0, 0)),
            pl.BlockSpec((1, D_HID), lambda i: (0, 0)),
            pl.BlockSpec((D_HID, D_IN), lambda i: (0, 0)),
        ],
        out_specs=pl.BlockSpec((RB, D_IN), lambda i: (i, 0)),
        out_shape=jax.ShapeDtypeStruct((N, D_IN), jnp.float32),
    )(x, W1T, b1.reshape(1, D_HID), W2T)


def _sc_scatter_body(z_hbm, b_hbm, zero_hbm, out_hbm, zbuf, ibuf, acc):
    c = lax.axis_index("c")
    s = lax.axis_index("s")
    w = c * NS + s

    zr0 = pl.multiple_of(s * SEG_PER_TILE, 8)
    pltpu.sync_copy(zero_hbm.at[pl.ds(zr0, SEG_PER_TILE)],
                    acc.at[pl.ds(zr0, SEG_PER_TILE)])

    @pl.when(s == NS - 1)
    def _zero_tail():
        t0 = pl.multiple_of(NS * SEG_PER_TILE, 8)
        pltpu.sync_copy(zero_hbm.at[pl.ds(t0, SEG_TAIL)],
                        acc.at[pl.ds(t0, SEG_TAIL)])

    plsc.subcore_barrier()

    row_base = w * RPW

    def body(j, carry):
        r0 = pl.multiple_of(row_base + j * K, 8)
        pltpu.sync_copy(z_hbm.at[pl.ds(r0, K)], zbuf)
        pltpu.sync_copy(b_hbm.at[pl.ds(r0, K)], ibuf)
        pltpu.sync_copy(zbuf, acc.at[ibuf], add=True)
        return carry

    lax.fori_loop(0, NCHUNK, body, 0)
    plsc.subcore_barrier()

    pltpu.sync_copy(acc.at[pl.ds(zr0, SEG_PER_TILE)],
                    out_hbm.at[c].at[pl.ds(zr0, SEG_PER_TILE)])

    @pl.when(s == NS - 1)
    def _write_tail():
        t0 = pl.multiple_of(NS * SEG_PER_TILE, 8)
        pltpu.sync_copy(acc.at[pl.ds(t0, SEG_TAIL)],
                        out_hbm.at[c].at[pl.ds(t0, SEG_TAIL)])


def _sc_scatter(z, bidx, zero):
    mesh = plsc.VectorSubcoreMesh(
        core_axis_name="c", subcore_axis_name="s", num_cores=NC, num_subcores=NS
    )
    return pl.kernel(
        _sc_scatter_body,
        out_type=jax.ShapeDtypeStruct((NC, S, D_IN), jnp.float32),
        mesh=mesh,
        scratch_types=[
            pltpu.VMEM((K, D_IN), jnp.float32),
            pltpu.VMEM((K,), jnp.int32),
            pltpu.VMEM_SHARED((S, D_IN), jnp.float32),
        ],
    )(z, bidx, zero)


def _combine_body(p_ref, b2_ref, o_ref):
    o_ref[...] = p_ref[0] + p_ref[1] + b2_ref[...]


def _combine(partials, b2):
    return pl.pallas_call(
        _combine_body,
        grid=(10,),
        in_specs=[
            pl.BlockSpec((NC, S // 10, D_IN), lambda i: (0, i, 0)),
            pl.BlockSpec((1, D_IN), lambda i: (0, 0)),
        ],
        out_specs=pl.BlockSpec((S // 10, D_IN), lambda i: (i, 0)),
        out_shape=jax.ShapeDtypeStruct((S, D_IN), jnp.float32),
    )(partials, b2.reshape(1, D_IN))


def kernel(x, batch, W1, b1, W2, b2):
    bidx = batch.astype(jnp.int32)
    z = _compute_z(x, W1.T, b1, W2.T)
    zero = jnp.zeros((S, D_IN), jnp.float32)
    partials = _sc_scatter(z, bidx, zero)
    return _combine(partials, b2)


# speedup vs baseline: 4.1071x; 1.3526x over previous
import jax
import jax.numpy as jnp
from jax import lax
from jax.experimental import pallas as pl
from jax.experimental.pallas import tpu as pltpu
from jax.experimental.pallas import tpu_sc as plsc

N = 320000
D_IN = 128
D_HID = 256
S = 10000

RB = 1280
NC = 2
NS = 16
NW = NC * NS
RPW = N // NW
K = 128
NK = RPW // K
KTAIL = RPW - NK * K
NBUF = 2
SEG_PER_TILE = 624
SEG_TAIL = S - NS * SEG_PER_TILE


def _z_body(x_ref, w1t_ref, b1_ref, w2t_ref, z_ref):
    h = jnp.dot(x_ref[...], w1t_ref[...], preferred_element_type=jnp.float32)
    phi = jnp.tanh(h + b1_ref[...])
    z_ref[...] = jnp.dot(phi, w2t_ref[...], preferred_element_type=jnp.float32)


def _compute_z(x, W1T, b1, W2T):
    return pl.pallas_call(
        _z_body,
        grid=(N // RB,),
        in_specs=[
            pl.BlockSpec((RB, D_IN), lambda i: (i, 0)),
            pl.BlockSpec((D_IN, D_HID), lambda i: (0, 0)),
            pl.BlockSpec((1, D_HID), lambda i: (0, 0)),
            pl.BlockSpec((D_HID, D_IN), lambda i: (0, 0)),
        ],
        out_specs=pl.BlockSpec((RB, D_IN), lambda i: (i, 0)),
        out_shape=jax.ShapeDtypeStruct((N, D_IN), jnp.float32),
    )(x, W1T, b1.reshape(1, D_HID), W2T)


def _sc_scatter_body(z_hbm, b_hbm, zero_hbm, out_hbm,
                     zb0, zb1, ib0, ib1, tbuf, acc, zs0, zs1, is0, is1):
    c = lax.axis_index("c")
    s = lax.axis_index("s")
    w = c * NS + s
    row_base = w * RPW
    zbufs = (zb0, zb1)
    ibufs = (ib0, ib1)
    zsems = (zs0, zs1)
    isems = (is0, is1)

    zr0 = pl.multiple_of(s * SEG_PER_TILE, 8)
    pltpu.sync_copy(zero_hbm.at[pl.ds(zr0, SEG_PER_TILE)],
                    acc.at[pl.ds(zr0, SEG_PER_TILE)])

    @pl.when(s == NS - 1)
    def _zero_tail():
        t0 = pl.multiple_of(NS * SEG_PER_TILE, 8)
        pltpu.sync_copy(zero_hbm.at[pl.ds(t0, SEG_TAIL)],
                        acc.at[pl.ds(t0, SEG_TAIL)])

    plsc.subcore_barrier()

    def zsrc(l):
        return z_hbm.at[pl.ds(pl.multiple_of(row_base + l * K, 8), K)]

    def isrc(l):
        return b_hbm.at[pl.ds(pl.multiple_of(row_base + l * K, 8), K)]

    for b in range(NBUF):
        pltpu.async_copy(zsrc(b), zbufs[b], zsems[b])
        pltpu.async_copy(isrc(b), ibufs[b], isems[b])

    def body(i, carry):
        for b in range(NBUF):
            l = i * NBUF + b
            pltpu.make_async_copy(zsrc(l), zbufs[b], zsems[b]).wait()
            pltpu.make_async_copy(isrc(l), ibufs[b], isems[b]).wait()
            pltpu.sync_copy(zbufs[b], acc.at[ibufs[b]], add=True)
            nl = l + NBUF

            @pl.when(nl < NK)
            def _next():
                pltpu.async_copy(zsrc(nl), zbufs[b], zsems[b])
                pltpu.async_copy(isrc(nl), ibufs[b], isems[b])

        return carry

    lax.fori_loop(0, NK // NBUF, body, 0)

    t0 = pl.multiple_of(row_base + NK * K, 8)
    pltpu.sync_copy(z_hbm.at[pl.ds(t0, KTAIL)], zb0.at[pl.ds(0, KTAIL)])
    pltpu.sync_copy(b_hbm.at[pl.ds(t0, KTAIL)], tbuf)
    pltpu.sync_copy(zb0.at[pl.ds(0, KTAIL)], acc.at[tbuf], add=True)

    plsc.subcore_barrier()

    pltpu.sync_copy(acc.at[pl.ds(zr0, SEG_PER_TILE)],
                    out_hbm.at[c].at[pl.ds(zr0, SEG_PER_TILE)])

    @pl.when(s == NS - 1)
    def _write_tail():
        t0w = pl.multiple_of(NS * SEG_PER_TILE, 8)
        pltpu.sync_copy(acc.at[pl.ds(t0w, SEG_TAIL)],
                        out_hbm.at[c].at[pl.ds(t0w, SEG_TAIL)])


def _sc_scatter(z, bidx, zero):
    mesh = plsc.VectorSubcoreMesh(
        core_axis_name="c", subcore_axis_name="s", num_cores=NC, num_subcores=NS
    )
    return pl.kernel(
        _sc_scatter_body,
        out_type=jax.ShapeDtypeStruct((NC, S, D_IN), jnp.float32),
        mesh=mesh,
        scratch_types=[
            pltpu.VMEM((K, D_IN), jnp.float32),
            pltpu.VMEM((K, D_IN), jnp.float32),
            pltpu.VMEM((K,), jnp.int32),
            pltpu.VMEM((K,), jnp.int32),
            pltpu.VMEM((KTAIL,), jnp.int32),
            pltpu.VMEM_SHARED((S, D_IN), jnp.float32),
            pltpu.SemaphoreType.DMA,
            pltpu.SemaphoreType.DMA,
            pltpu.SemaphoreType.DMA,
            pltpu.SemaphoreType.DMA,
        ],
    )(z, bidx, zero)


def _combine_body(p_ref, b2_ref, o_ref):
    o_ref[...] = p_ref[0] + p_ref[1] + b2_ref[...]


def _combine(partials, b2):
    return pl.pallas_call(
        _combine_body,
        grid=(10,),
        in_specs=[
            pl.BlockSpec((NC, S // 10, D_IN), lambda i: (0, i, 0)),
            pl.BlockSpec((1, D_IN), lambda i: (0, 0)),
        ],
        out_specs=pl.BlockSpec((S // 10, D_IN), lambda i: (i, 0)),
        out_shape=jax.ShapeDtypeStruct((S, D_IN), jnp.float32),
    )(partials, b2.reshape(1, D_IN))


def kernel(x, batch, W1, b1, W2, b2):
    bidx = batch.astype(jnp.int32)
    z = _compute_z(x, W1.T, b1, W2.T)
    zero = jnp.zeros((S, D_IN), jnp.float32)
    partials = _sc_scatter(z, bidx, zero)
    return _combine(partials, b2)


# speedup vs baseline: 4.1080x; 1.0002x over previous
import jax
import jax.numpy as jnp
from jax import lax
from jax.experimental import pallas as pl
from jax.experimental.pallas import tpu as pltpu
from jax.experimental.pallas import tpu_sc as plsc

N = 320000
D_IN = 128
D_HID = 256
S = 10000

RB = 1280
NC = 2
NS = 16
NW = NC * NS
RPW = N // NW
K = 128
NK = RPW // K
KTAIL = RPW - NK * K
NBUF = 2
SEG_PER_TILE = 624
SEG_TAIL = S - NS * SEG_PER_TILE


def _z_body(x_ref, w1t_ref, b1_ref, w2t_ref, z_ref):
    xb = x_ref[...].astype(jnp.bfloat16)
    h = jnp.dot(xb, w1t_ref[...], preferred_element_type=jnp.float32)
    phi = jnp.tanh(h + b1_ref[...])
    z_ref[...] = jnp.dot(phi.astype(jnp.bfloat16), w2t_ref[...],
                         preferred_element_type=jnp.float32)


def _compute_z(x, W1T, b1, W2T):
    return pl.pallas_call(
        _z_body,
        grid=(N // RB,),
        in_specs=[
            pl.BlockSpec((RB, D_IN), lambda i: (i, 0)),
            pl.BlockSpec((D_IN, D_HID), lambda i: (0, 0)),
            pl.BlockSpec((1, D_HID), lambda i: (0, 0)),
            pl.BlockSpec((D_HID, D_IN), lambda i: (0, 0)),
        ],
        out_specs=pl.BlockSpec((RB, D_IN), lambda i: (i, 0)),
        out_shape=jax.ShapeDtypeStruct((N, D_IN), jnp.float32),
    )(x, W1T, b1.reshape(1, D_HID), W2T)


def _sc_scatter_body(z_hbm, b_hbm, zero_hbm, out_hbm,
                     zb0, zb1, ib0, ib1, tbuf, acc, zs0, zs1, is0, is1):
    c = lax.axis_index("c")
    s = lax.axis_index("s")
    w = c * NS + s
    row_base = w * RPW
    zbufs = (zb0, zb1)
    ibufs = (ib0, ib1)
    zsems = (zs0, zs1)
    isems = (is0, is1)

    zr0 = pl.multiple_of(s * SEG_PER_TILE, 8)
    pltpu.sync_copy(zero_hbm.at[pl.ds(zr0, SEG_PER_TILE)],
                    acc.at[pl.ds(zr0, SEG_PER_TILE)])

    @pl.when(s == NS - 1)
    def _zero_tail():
        t0 = pl.multiple_of(NS * SEG_PER_TILE, 8)
        pltpu.sync_copy(zero_hbm.at[pl.ds(t0, SEG_TAIL)],
                        acc.at[pl.ds(t0, SEG_TAIL)])

    plsc.subcore_barrier()

    def zsrc(l):
        return z_hbm.at[pl.ds(pl.multiple_of(row_base + l * K, 8), K)]

    def isrc(l):
        return b_hbm.at[pl.ds(pl.multiple_of(row_base + l * K, 8), K)]

    for b in range(NBUF):
        pltpu.async_copy(zsrc(b), zbufs[b], zsems[b])
        pltpu.async_copy(isrc(b), ibufs[b], isems[b])

    def body(i, carry):
        for b in range(NBUF):
            l = i * NBUF + b
            pltpu.make_async_copy(zsrc(l), zbufs[b], zsems[b]).wait()
            pltpu.make_async_copy(isrc(l), ibufs[b], isems[b]).wait()
            pltpu.sync_copy(zbufs[b], acc.at[ibufs[b]], add=True)
            nl = l + NBUF

            @pl.when(nl < NK)
            def _next():
                pltpu.async_copy(zsrc(nl), zbufs[b], zsems[b])
                pltpu.async_copy(isrc(nl), ibufs[b], isems[b])

        return carry

    lax.fori_loop(0, NK // NBUF, body, 0)

    t0 = pl.multiple_of(row_base + NK * K, 8)
    pltpu.sync_copy(z_hbm.at[pl.ds(t0, KTAIL)], zb0.at[pl.ds(0, KTAIL)])
    pltpu.sync_copy(b_hbm.at[pl.ds(t0, KTAIL)], tbuf)
    pltpu.sync_copy(zb0.at[pl.ds(0, KTAIL)], acc.at[tbuf], add=True)

    plsc.subcore_barrier()

    pltpu.sync_copy(acc.at[pl.ds(zr0, SEG_PER_TILE)],
                    out_hbm.at[c].at[pl.ds(zr0, SEG_PER_TILE)])

    @pl.when(s == NS - 1)
    def _write_tail():
        t0w = pl.multiple_of(NS * SEG_PER_TILE, 8)
        pltpu.sync_copy(acc.at[pl.ds(t0w, SEG_TAIL)],
                        out_hbm.at[c].at[pl.ds(t0w, SEG_TAIL)])


def _sc_scatter(z, bidx, zero):
    mesh = plsc.VectorSubcoreMesh(
        core_axis_name="c", subcore_axis_name="s", num_cores=NC, num_subcores=NS
    )
    return pl.kernel(
        _sc_scatter_body,
        out_type=jax.ShapeDtypeStruct((NC, S, D_IN), jnp.float32),
        mesh=mesh,
        scratch_types=[
            pltpu.VMEM((K, D_IN), jnp.float32),
            pltpu.VMEM((K, D_IN), jnp.float32),
            pltpu.VMEM((K,), jnp.int32),
            pltpu.VMEM((K,), jnp.int32),
            pltpu.VMEM((KTAIL,), jnp.int32),
            pltpu.VMEM_SHARED((S, D_IN), jnp.float32),
            pltpu.SemaphoreType.DMA,
            pltpu.SemaphoreType.DMA,
            pltpu.SemaphoreType.DMA,
            pltpu.SemaphoreType.DMA,
        ],
    )(z, bidx, zero)


def _combine_body(p_ref, b2_ref, o_ref):
    o_ref[...] = p_ref[0] + p_ref[1] + b2_ref[...]


def _combine(partials, b2):
    return pl.pallas_call(
        _combine_body,
        grid=(10,),
        in_specs=[
            pl.BlockSpec((NC, S // 10, D_IN), lambda i: (0, i, 0)),
            pl.BlockSpec((1, D_IN), lambda i: (0, 0)),
        ],
        out_specs=pl.BlockSpec((S // 10, D_IN), lambda i: (i, 0)),
        out_shape=jax.ShapeDtypeStruct((S, D_IN), jnp.float32),
    )(partials, b2.reshape(1, D_IN))


def kernel(x, batch, W1, b1, W2, b2):
    bidx = batch.astype(jnp.int32)
    z = _compute_z(x, W1.T.astype(jnp.bfloat16), b1, W2.T.astype(jnp.bfloat16))
    zero = jnp.zeros((S, D_IN), jnp.float32)
    partials = _sc_scatter(z, bidx, zero)
    return _combine(partials, b2)


# speedup vs baseline: 5.0937x; 1.2399x over previous
import jax
import jax.numpy as jnp
from jax import lax
from jax.experimental import pallas as pl
from jax.experimental.pallas import tpu as pltpu
from jax.experimental.pallas import tpu_sc as plsc

N = 320000
D_IN = 128
D_HID = 256
S = 10000

RB = 2560
NC = 2
NS = 16
NW = NC * NS
RPW = N // NW
K = 128
NK = RPW // K
KTAIL = RPW - NK * K
NBUF = 2
SEG_PER_TILE = 624
SEG_TAIL = S - NS * SEG_PER_TILE


def _z_body(x_ref, w1t_ref, b1_ref, w2t_ref, z_ref):
    xb = x_ref[...].astype(jnp.bfloat16)
    h = jnp.dot(xb, w1t_ref[...], preferred_element_type=jnp.float32)
    phi = jnp.tanh(h + b1_ref[...])
    z_ref[...] = jnp.dot(phi.astype(jnp.bfloat16), w2t_ref[...],
                         preferred_element_type=jnp.float32)


def _compute_z(x, W1T, b1, W2T):
    return pl.pallas_call(
        _z_body,
        grid=(N // RB,),
        in_specs=[
            pl.BlockSpec((RB, D_IN), lambda i: (i, 0)),
            pl.BlockSpec((D_IN, D_HID), lambda i: (0, 0)),
            pl.BlockSpec((1, D_HID), lambda i: (0, 0)),
            pl.BlockSpec((D_HID, D_IN), lambda i: (0, 0)),
        ],
        out_specs=pl.BlockSpec((RB, D_IN), lambda i: (i, 0)),
        out_shape=jax.ShapeDtypeStruct((N, D_IN), jnp.float32),
    )(x, W1T, b1.reshape(1, D_HID), W2T)


def _sc_scatter_body(z_hbm, b_hbm, zero_hbm, out_hbm,
                     zb0, zb1, ib0, ib1, tbuf, acc, zs0, zs1, is0, is1):
    c = lax.axis_index("c")
    s = lax.axis_index("s")
    w = c * NS + s
    row_base = w * RPW
    zbufs = (zb0, zb1)
    ibufs = (ib0, ib1)
    zsems = (zs0, zs1)
    isems = (is0, is1)

    zr0 = pl.multiple_of(s * SEG_PER_TILE, 8)
    pltpu.sync_copy(zero_hbm.at[pl.ds(zr0, SEG_PER_TILE)],
                    acc.at[pl.ds(zr0, SEG_PER_TILE)])

    @pl.when(s == NS - 1)
    def _zero_tail():
        t0 = pl.multiple_of(NS * SEG_PER_TILE, 8)
        pltpu.sync_copy(zero_hbm.at[pl.ds(t0, SEG_TAIL)],
                        acc.at[pl.ds(t0, SEG_TAIL)])

    plsc.subcore_barrier()

    def zsrc(l):
        return z_hbm.at[pl.ds(pl.multiple_of(row_base + l * K, 8), K)]

    def isrc(l):
        return b_hbm.at[pl.ds(pl.multiple_of(row_base + l * K, 8), K)]

    for b in range(NBUF):
        pltpu.async_copy(zsrc(b), zbufs[b], zsems[b])
        pltpu.async_copy(isrc(b), ibufs[b], isems[b])

    def body(i, carry):
        for b in range(NBUF):
            l = i * NBUF + b
            pltpu.make_async_copy(zsrc(l), zbufs[b], zsems[b]).wait()
            pltpu.make_async_copy(isrc(l), ibufs[b], isems[b]).wait()
            pltpu.sync_copy(zbufs[b], acc.at[ibufs[b]], add=True)
            nl = l + NBUF

            @pl.when(nl < NK)
            def _next():
                pltpu.async_copy(zsrc(nl), zbufs[b], zsems[b])
                pltpu.async_copy(isrc(nl), ibufs[b], isems[b])

        return carry

    lax.fori_loop(0, NK // NBUF, body, 0)

    t0 = pl.multiple_of(row_base + NK * K, 8)
    pltpu.sync_copy(z_hbm.at[pl.ds(t0, KTAIL)], zb0.at[pl.ds(0, KTAIL)])
    pltpu.sync_copy(b_hbm.at[pl.ds(t0, KTAIL)], tbuf)
    pltpu.sync_copy(zb0.at[pl.ds(0, KTAIL)], acc.at[tbuf], add=True)

    plsc.subcore_barrier()

    pltpu.sync_copy(acc.at[pl.ds(zr0, SEG_PER_TILE)],
                    out_hbm.at[c].at[pl.ds(zr0, SEG_PER_TILE)])

    @pl.when(s == NS - 1)
    def _write_tail():
        t0w = pl.multiple_of(NS * SEG_PER_TILE, 8)
        pltpu.sync_copy(acc.at[pl.ds(t0w, SEG_TAIL)],
                        out_hbm.at[c].at[pl.ds(t0w, SEG_TAIL)])


def _sc_scatter(z, bidx, zero):
    mesh = plsc.VectorSubcoreMesh(
        core_axis_name="c", subcore_axis_name="s", num_cores=NC, num_subcores=NS
    )
    return pl.kernel(
        _sc_scatter_body,
        out_type=jax.ShapeDtypeStruct((NC, S, D_IN), jnp.float32),
        mesh=mesh,
        scratch_types=[
            pltpu.VMEM((K, D_IN), jnp.float32),
            pltpu.VMEM((K, D_IN), jnp.float32),
            pltpu.VMEM((K,), jnp.int32),
            pltpu.VMEM((K,), jnp.int32),
            pltpu.VMEM((KTAIL,), jnp.int32),
            pltpu.VMEM_SHARED((S, D_IN), jnp.float32),
            pltpu.SemaphoreType.DMA,
            pltpu.SemaphoreType.DMA,
            pltpu.SemaphoreType.DMA,
            pltpu.SemaphoreType.DMA,
        ],
    )(z, bidx, zero)


def _combine_body(p_ref, b2_ref, o_ref):
    o_ref[...] = p_ref[0] + p_ref[1] + b2_ref[...]


def _combine(partials, b2):
    return pl.pallas_call(
        _combine_body,
        grid=(10,),
        in_specs=[
            pl.BlockSpec((NC, S // 10, D_IN), lambda i: (0, i, 0)),
            pl.BlockSpec((1, D_IN), lambda i: (0, 0)),
        ],
        out_specs=pl.BlockSpec((S // 10, D_IN), lambda i: (i, 0)),
        out_shape=jax.ShapeDtypeStruct((S, D_IN), jnp.float32),
    )(partials, b2.reshape(1, D_IN))


def kernel(x, batch, W1, b1, W2, b2):
    bidx = batch.astype(jnp.int32)
    z = _compute_z(x, W1.T.astype(jnp.bfloat16), b1, W2.T.astype(jnp.bfloat16))
    zero = jnp.zeros((S, D_IN), jnp.float32)
    partials = _sc_scatter(z, bidx, zero)
    return _combine(partials, b2)


# speedup vs baseline: 6.0080x; 1.1795x over previous
import jax
import jax.numpy as jnp
from jax import lax
from jax.experimental import pallas as pl
from jax.experimental.pallas import tpu as pltpu
from jax.experimental.pallas import tpu_sc as plsc

N = 320000
D_IN = 128
D_HID = 256
S = 10000

RB = 6400
NC = 2
NS = 16
NW = NC * NS
RPW = N // NW
K = 128
NK = RPW // K
KTAIL = RPW - NK * K
NBUF = 2
SEG_PER_TILE = 624
SEG_TAIL = S - NS * SEG_PER_TILE


def _z_body(x_ref, w1t_ref, b1_ref, w2t_ref, z_ref):
    xb = x_ref[...].astype(jnp.bfloat16)
    h = jnp.dot(xb, w1t_ref[...], preferred_element_type=jnp.float32)
    phi = jnp.tanh(h + b1_ref[...])
    z_ref[...] = jnp.dot(phi.astype(jnp.bfloat16), w2t_ref[...],
                         preferred_element_type=jnp.float32)


def _compute_z(x, W1T, b1, W2T):
    return pl.pallas_call(
        _z_body,
        grid=(N // RB,),
        in_specs=[
            pl.BlockSpec((RB, D_IN), lambda i: (i, 0)),
            pl.BlockSpec((D_IN, D_HID), lambda i: (0, 0)),
            pl.BlockSpec((1, D_HID), lambda i: (0, 0)),
            pl.BlockSpec((D_HID, D_IN), lambda i: (0, 0)),
        ],
        out_specs=pl.BlockSpec((RB, D_IN), lambda i: (i, 0)),
        out_shape=jax.ShapeDtypeStruct((N, D_IN), jnp.float32),
    )(x, W1T, b1.reshape(1, D_HID), W2T)


def _sc_scatter_body(z_hbm, b_hbm, zero_hbm, out_hbm,
                     zb0, zb1, ib0, ib1, tbuf, acc, zs0, zs1, is0, is1):
    c = lax.axis_index("c")
    s = lax.axis_index("s")
    w = c * NS + s
    row_base = w * RPW
    zbufs = (zb0, zb1)
    ibufs = (ib0, ib1)
    zsems = (zs0, zs1)
    isems = (is0, is1)

    zr0 = pl.multiple_of(s * SEG_PER_TILE, 8)
    pltpu.sync_copy(zero_hbm.at[pl.ds(zr0, SEG_PER_TILE)],
                    acc.at[pl.ds(zr0, SEG_PER_TILE)])

    @pl.when(s == NS - 1)
    def _zero_tail():
        t0 = pl.multiple_of(NS * SEG_PER_TILE, 8)
        pltpu.sync_copy(zero_hbm.at[pl.ds(t0, SEG_TAIL)],
                        acc.at[pl.ds(t0, SEG_TAIL)])

    plsc.subcore_barrier()

    def zsrc(l):
        return z_hbm.at[pl.ds(pl.multiple_of(row_base + l * K, 8), K)]

    def isrc(l):
        return b_hbm.at[pl.ds(pl.multiple_of(row_base + l * K, 8), K)]

    for b in range(NBUF):
        pltpu.async_copy(zsrc(b), zbufs[b], zsems[b])
        pltpu.async_copy(isrc(b), ibufs[b], isems[b])

    def body(i, carry):
        for b in range(NBUF):
            l = i * NBUF + b
            pltpu.make_async_copy(zsrc(l), zbufs[b], zsems[b]).wait()
            pltpu.make_async_copy(isrc(l), ibufs[b], isems[b]).wait()
            pltpu.sync_copy(zbufs[b], acc.at[ibufs[b]], add=True)
            nl = l + NBUF

            @pl.when(nl < NK)
            def _next():
                pltpu.async_copy(zsrc(nl), zbufs[b], zsems[b])
                pltpu.async_copy(isrc(nl), ibufs[b], isems[b])

        return carry

    lax.fori_loop(0, NK // NBUF, body, 0)

    t0 = pl.multiple_of(row_base + NK * K, 8)
    pltpu.sync_copy(z_hbm.at[pl.ds(t0, KTAIL)], zb0.at[pl.ds(0, KTAIL)])
    pltpu.sync_copy(b_hbm.at[pl.ds(t0, KTAIL)], tbuf)
    pltpu.sync_copy(zb0.at[pl.ds(0, KTAIL)], acc.at[tbuf], add=True)

    plsc.subcore_barrier()

    pltpu.sync_copy(acc.at[pl.ds(zr0, SEG_PER_TILE)],
                    out_hbm.at[c].at[pl.ds(zr0, SEG_PER_TILE)])

    @pl.when(s == NS - 1)
    def _write_tail():
        t0w = pl.multiple_of(NS * SEG_PER_TILE, 8)
        pltpu.sync_copy(acc.at[pl.ds(t0w, SEG_TAIL)],
                        out_hbm.at[c].at[pl.ds(t0w, SEG_TAIL)])


def _sc_scatter(z, bidx, zero):
    mesh = plsc.VectorSubcoreMesh(
        core_axis_name="c", subcore_axis_name="s", num_cores=NC, num_subcores=NS
    )
    return pl.kernel(
        _sc_scatter_body,
        out_type=jax.ShapeDtypeStruct((NC, S, D_IN), jnp.float32),
        mesh=mesh,
        scratch_types=[
            pltpu.VMEM((K, D_IN), jnp.float32),
            pltpu.VMEM((K, D_IN), jnp.float32),
            pltpu.VMEM((K,), jnp.int32),
            pltpu.VMEM((K,), jnp.int32),
            pltpu.VMEM((KTAIL,), jnp.int32),
            pltpu.VMEM_SHARED((S, D_IN), jnp.float32),
            pltpu.SemaphoreType.DMA,
            pltpu.SemaphoreType.DMA,
            pltpu.SemaphoreType.DMA,
            pltpu.SemaphoreType.DMA,
        ],
    )(z, bidx, zero)


def _combine_body(p_ref, b2_ref, o_ref):
    o_ref[...] = p_ref[0] + p_ref[1] + b2_ref[...]


def _combine(partials, b2):
    return pl.pallas_call(
        _combine_body,
        grid=(10,),
        in_specs=[
            pl.BlockSpec((NC, S // 10, D_IN), lambda i: (0, i, 0)),
            pl.BlockSpec((1, D_IN), lambda i: (0, 0)),
        ],
        out_specs=pl.BlockSpec((S // 10, D_IN), lambda i: (i, 0)),
        out_shape=jax.ShapeDtypeStruct((S, D_IN), jnp.float32),
    )(partials, b2.reshape(1, D_IN))


def kernel(x, batch, W1, b1, W2, b2):
    bidx = batch.astype(jnp.int32)
    z = _compute_z(x, W1.T.astype(jnp.bfloat16), b1, W2.T.astype(jnp.bfloat16))
    zero = jnp.zeros((S, D_IN), jnp.float32)
    partials = _sc_scatter(z, bidx, zero)
    return _combine(partials, b2)


# speedup vs baseline: 6.1812x; 1.0288x over previous
import jax
import jax.numpy as jnp
from jax import lax
from jax.experimental import pallas as pl
from jax.experimental.pallas import tpu as pltpu
from jax.experimental.pallas import tpu_sc as plsc

N = 320000
D_IN = 128
D_HID = 256
S = 10000

RB = 8000
NC = 2
NS = 16
NW = NC * NS
RPW = N // NW
K = 128
NK = RPW // K
KTAIL = RPW - NK * K
NBUF = 2
SEG_PER_TILE = 624
SEG_TAIL = S - NS * SEG_PER_TILE


def _z_body(x_ref, w1t_ref, b1_ref, w2t_ref, z_ref):
    xb = x_ref[...].astype(jnp.bfloat16)
    h = jnp.dot(xb, w1t_ref[...], preferred_element_type=jnp.float32)
    phi = jnp.tanh(h + b1_ref[...])
    z_ref[...] = jnp.dot(phi.astype(jnp.bfloat16), w2t_ref[...],
                         preferred_element_type=jnp.float32)


def _compute_z(x, W1T, b1, W2T):
    return pl.pallas_call(
        _z_body,
        grid=(N // RB,),
        in_specs=[
            pl.BlockSpec((RB, D_IN), lambda i: (i, 0)),
            pl.BlockSpec((D_IN, D_HID), lambda i: (0, 0)),
            pl.BlockSpec((1, D_HID), lambda i: (0, 0)),
            pl.BlockSpec((D_HID, D_IN), lambda i: (0, 0)),
        ],
        out_specs=pl.BlockSpec((RB, D_IN), lambda i: (i, 0)),
        out_shape=jax.ShapeDtypeStruct((N, D_IN), jnp.float32),
    )(x, W1T, b1.reshape(1, D_HID), W2T)


def _sc_scatter_body(z_hbm, b_hbm, zero_hbm, out_hbm,
                     zb0, zb1, ib0, ib1, tbuf, acc, zs0, zs1, is0, is1):
    c = lax.axis_index("c")
    s = lax.axis_index("s")
    w = c * NS + s
    row_base = w * RPW
    zbufs = (zb0, zb1)
    ibufs = (ib0, ib1)
    zsems = (zs0, zs1)
    isems = (is0, is1)

    zr0 = pl.multiple_of(s * SEG_PER_TILE, 8)
    pltpu.sync_copy(zero_hbm.at[pl.ds(zr0, SEG_PER_TILE)],
                    acc.at[pl.ds(zr0, SEG_PER_TILE)])

    @pl.when(s == NS - 1)
    def _zero_tail():
        t0 = pl.multiple_of(NS * SEG_PER_TILE, 8)
        pltpu.sync_copy(zero_hbm.at[pl.ds(t0, SEG_TAIL)],
                        acc.at[pl.ds(t0, SEG_TAIL)])

    plsc.subcore_barrier()

    def zsrc(l):
        return z_hbm.at[pl.ds(pl.multiple_of(row_base + l * K, 8), K)]

    def isrc(l):
        return b_hbm.at[pl.ds(pl.multiple_of(row_base + l * K, 8), K)]

    for b in range(NBUF):
        pltpu.async_copy(zsrc(b), zbufs[b], zsems[b])
        pltpu.async_copy(isrc(b), ibufs[b], isems[b])

    def body(i, carry):
        for b in range(NBUF):
            l = i * NBUF + b
            pltpu.make_async_copy(zsrc(l), zbufs[b], zsems[b]).wait()
            pltpu.make_async_copy(isrc(l), ibufs[b], isems[b]).wait()
            pltpu.sync_copy(zbufs[b], acc.at[ibufs[b]], add=True)
            nl = l + NBUF

            @pl.when(nl < NK)
            def _next():
                pltpu.async_copy(zsrc(nl), zbufs[b], zsems[b])
                pltpu.async_copy(isrc(nl), ibufs[b], isems[b])

        return carry

    lax.fori_loop(0, NK // NBUF, body, 0)

    t0 = pl.multiple_of(row_base + NK * K, 8)
    pltpu.sync_copy(z_hbm.at[pl.ds(t0, KTAIL)], zb0.at[pl.ds(0, KTAIL)])
    pltpu.sync_copy(b_hbm.at[pl.ds(t0, KTAIL)], tbuf)
    pltpu.sync_copy(zb0.at[pl.ds(0, KTAIL)], acc.at[tbuf], add=True)

    plsc.subcore_barrier()

    pltpu.sync_copy(acc.at[pl.ds(zr0, SEG_PER_TILE)],
                    out_hbm.at[c].at[pl.ds(zr0, SEG_PER_TILE)])

    @pl.when(s == NS - 1)
    def _write_tail():
        t0w = pl.multiple_of(NS * SEG_PER_TILE, 8)
        pltpu.sync_copy(acc.at[pl.ds(t0w, SEG_TAIL)],
                        out_hbm.at[c].at[pl.ds(t0w, SEG_TAIL)])


def _sc_scatter(z, bidx, zero):
    mesh = plsc.VectorSubcoreMesh(
        core_axis_name="c", subcore_axis_name="s", num_cores=NC, num_subcores=NS
    )
    return pl.kernel(
        _sc_scatter_body,
        out_type=jax.ShapeDtypeStruct((NC, S, D_IN), jnp.float32),
        mesh=mesh,
        scratch_types=[
            pltpu.VMEM((K, D_IN), jnp.float32),
            pltpu.VMEM((K, D_IN), jnp.float32),
            pltpu.VMEM((K,), jnp.int32),
            pltpu.VMEM((K,), jnp.int32),
            pltpu.VMEM((KTAIL,), jnp.int32),
            pltpu.VMEM_SHARED((S, D_IN), jnp.float32),
            pltpu.SemaphoreType.DMA,
            pltpu.SemaphoreType.DMA,
            pltpu.SemaphoreType.DMA,
            pltpu.SemaphoreType.DMA,
        ],
    )(z, bidx, zero)


def _combine_body(p_ref, b2_ref, o_ref):
    o_ref[...] = p_ref[0] + p_ref[1] + b2_ref[...]


def _combine(partials, b2):
    return pl.pallas_call(
        _combine_body,
        grid=(10,),
        in_specs=[
            pl.BlockSpec((NC, S // 10, D_IN), lambda i: (0, i, 0)),
            pl.BlockSpec((1, D_IN), lambda i: (0, 0)),
        ],
        out_specs=pl.BlockSpec((S // 10, D_IN), lambda i: (i, 0)),
        out_shape=jax.ShapeDtypeStruct((S, D_IN), jnp.float32),
    )(partials, b2.reshape(1, D_IN))


def kernel(x, batch, W1, b1, W2, b2):
    bidx = batch.astype(jnp.int32)
    z = _compute_z(x, W1.T.astype(jnp.bfloat16), b1, W2.T.astype(jnp.bfloat16))
    zero = jnp.zeros((S, D_IN), jnp.float32)
    partials = _sc_scatter(z, bidx, zero)
    return _combine(partials, b2)


# speedup vs baseline: 6.4084x; 1.0368x over previous
import jax
import jax.numpy as jnp
from jax import lax
from jax.experimental import pallas as pl
from jax.experimental.pallas import tpu as pltpu
from jax.experimental.pallas import tpu_sc as plsc

N = 320000
D_IN = 128
D_HID = 256
S = 10000

RB = 12800
NC = 2
NS = 16
NW = NC * NS
RPW = N // NW
K = 128
NK = RPW // K
KTAIL = RPW - NK * K
NBUF = 2
SEG_PER_TILE = 624
SEG_TAIL = S - NS * SEG_PER_TILE


def _z_body(x_ref, w1t_ref, b1_ref, w2t_ref, z_ref):
    xb = x_ref[...].astype(jnp.bfloat16)
    h = jnp.dot(xb, w1t_ref[...], preferred_element_type=jnp.float32)
    phi = jnp.tanh(h + b1_ref[...])
    z_ref[...] = jnp.dot(phi.astype(jnp.bfloat16), w2t_ref[...],
                         preferred_element_type=jnp.float32)


def _compute_z(x, W1T, b1, W2T):
    return pl.pallas_call(
        _z_body,
        grid=(N // RB,),
        in_specs=[
            pl.BlockSpec((RB, D_IN), lambda i: (i, 0)),
            pl.BlockSpec((D_IN, D_HID), lambda i: (0, 0)),
            pl.BlockSpec((1, D_HID), lambda i: (0, 0)),
            pl.BlockSpec((D_HID, D_IN), lambda i: (0, 0)),
        ],
        out_specs=pl.BlockSpec((RB, D_IN), lambda i: (i, 0)),
        out_shape=jax.ShapeDtypeStruct((N, D_IN), jnp.float32),
    )(x, W1T, b1.reshape(1, D_HID), W2T)


def _sc_scatter_body(z_hbm, b_hbm, zero_hbm, out_hbm,
                     zb0, zb1, ib0, ib1, tbuf, acc, zs0, zs1, is0, is1):
    c = lax.axis_index("c")
    s = lax.axis_index("s")
    w = c * NS + s
    row_base = w * RPW
    zbufs = (zb0, zb1)
    ibufs = (ib0, ib1)
    zsems = (zs0, zs1)
    isems = (is0, is1)

    zr0 = pl.multiple_of(s * SEG_PER_TILE, 8)
    pltpu.sync_copy(zero_hbm.at[pl.ds(zr0, SEG_PER_TILE)],
                    acc.at[pl.ds(zr0, SEG_PER_TILE)])

    @pl.when(s == NS - 1)
    def _zero_tail():
        t0 = pl.multiple_of(NS * SEG_PER_TILE, 8)
        pltpu.sync_copy(zero_hbm.at[pl.ds(t0, SEG_TAIL)],
                        acc.at[pl.ds(t0, SEG_TAIL)])

    plsc.subcore_barrier()

    def zsrc(l):
        return z_hbm.at[pl.ds(pl.multiple_of(row_base + l * K, 8), K)]

    def isrc(l):
        return b_hbm.at[pl.ds(pl.multiple_of(row_base + l * K, 8), K)]

    for b in range(NBUF):
        pltpu.async_copy(zsrc(b), zbufs[b], zsems[b])
        pltpu.async_copy(isrc(b), ibufs[b], isems[b])

    def body(i, carry):
        for b in range(NBUF):
            l = i * NBUF + b
            pltpu.make_async_copy(zsrc(l), zbufs[b], zsems[b]).wait()
            pltpu.make_async_copy(isrc(l), ibufs[b], isems[b]).wait()
            pltpu.sync_copy(zbufs[b], acc.at[ibufs[b]], add=True)
            nl = l + NBUF

            @pl.when(nl < NK)
            def _next():
                pltpu.async_copy(zsrc(nl), zbufs[b], zsems[b])
                pltpu.async_copy(isrc(nl), ibufs[b], isems[b])

        return carry

    lax.fori_loop(0, NK // NBUF, body, 0)

    t0 = pl.multiple_of(row_base + NK * K, 8)
    pltpu.sync_copy(z_hbm.at[pl.ds(t0, KTAIL)], zb0.at[pl.ds(0, KTAIL)])
    pltpu.sync_copy(b_hbm.at[pl.ds(t0, KTAIL)], tbuf)
    pltpu.sync_copy(zb0.at[pl.ds(0, KTAIL)], acc.at[tbuf], add=True)

    plsc.subcore_barrier()

    pltpu.sync_copy(acc.at[pl.ds(zr0, SEG_PER_TILE)],
                    out_hbm.at[c].at[pl.ds(zr0, SEG_PER_TILE)])

    @pl.when(s == NS - 1)
    def _write_tail():
        t0w = pl.multiple_of(NS * SEG_PER_TILE, 8)
        pltpu.sync_copy(acc.at[pl.ds(t0w, SEG_TAIL)],
                        out_hbm.at[c].at[pl.ds(t0w, SEG_TAIL)])


def _sc_scatter(z, bidx, zero):
    mesh = plsc.VectorSubcoreMesh(
        core_axis_name="c", subcore_axis_name="s", num_cores=NC, num_subcores=NS
    )
    return pl.kernel(
        _sc_scatter_body,
        out_type=jax.ShapeDtypeStruct((NC, S, D_IN), jnp.float32),
        mesh=mesh,
        scratch_types=[
            pltpu.VMEM((K, D_IN), jnp.float32),
            pltpu.VMEM((K, D_IN), jnp.float32),
            pltpu.VMEM((K,), jnp.int32),
            pltpu.VMEM((K,), jnp.int32),
            pltpu.VMEM((KTAIL,), jnp.int32),
            pltpu.VMEM_SHARED((S, D_IN), jnp.float32),
            pltpu.SemaphoreType.DMA,
            pltpu.SemaphoreType.DMA,
            pltpu.SemaphoreType.DMA,
            pltpu.SemaphoreType.DMA,
        ],
    )(z, bidx, zero)


def _combine_body(p_ref, b2_ref, o_ref):
    o_ref[...] = p_ref[0] + p_ref[1] + b2_ref[...]


def _combine(partials, b2):
    return pl.pallas_call(
        _combine_body,
        grid=(10,),
        in_specs=[
            pl.BlockSpec((NC, S // 10, D_IN), lambda i: (0, i, 0)),
            pl.BlockSpec((1, D_IN), lambda i: (0, 0)),
        ],
        out_specs=pl.BlockSpec((S // 10, D_IN), lambda i: (i, 0)),
        out_shape=jax.ShapeDtypeStruct((S, D_IN), jnp.float32),
    )(partials, b2.reshape(1, D_IN))


def kernel(x, batch, W1, b1, W2, b2):
    bidx = batch.astype(jnp.int32)
    z = _compute_z(x, W1.T.astype(jnp.bfloat16), b1, W2.T.astype(jnp.bfloat16))
    zero = jnp.zeros((S, D_IN), jnp.float32)
    partials = _sc_scatter(z, bidx, zero)
    return _combine(partials, b2)


# speedup vs baseline: 6.4984x; 1.0140x over previous
import jax
import jax.numpy as jnp
from jax import lax
from jax.experimental import pallas as pl
from jax.experimental.pallas import tpu as pltpu
from jax.experimental.pallas import tpu_sc as plsc

N = 320000
D_IN = 128
D_HID = 256
S = 10000

RB = 16000
NC = 2
NS = 16
NW = NC * NS
RPW = N // NW
K = 128
NK = RPW // K
KTAIL = RPW - NK * K
NBUF = 2
SEG_PER_TILE = 624
SEG_TAIL = S - NS * SEG_PER_TILE


def _z_body(x_ref, w1t_ref, b1_ref, w2t_ref, z_ref):
    xb = x_ref[...].astype(jnp.bfloat16)
    h = jnp.dot(xb, w1t_ref[...], preferred_element_type=jnp.float32)
    phi = jnp.tanh(h + b1_ref[...])
    z_ref[...] = jnp.dot(phi.astype(jnp.bfloat16), w2t_ref[...],
                         preferred_element_type=jnp.float32)


def _compute_z(x, W1T, b1, W2T):
    return pl.pallas_call(
        _z_body,
        grid=(N // RB,),
        in_specs=[
            pl.BlockSpec((RB, D_IN), lambda i: (i, 0)),
            pl.BlockSpec((D_IN, D_HID), lambda i: (0, 0)),
            pl.BlockSpec((1, D_HID), lambda i: (0, 0)),
            pl.BlockSpec((D_HID, D_IN), lambda i: (0, 0)),
        ],
        out_specs=pl.BlockSpec((RB, D_IN), lambda i: (i, 0)),
        out_shape=jax.ShapeDtypeStruct((N, D_IN), jnp.float32),
    )(x, W1T, b1.reshape(1, D_HID), W2T)


def _sc_scatter_body(z_hbm, b_hbm, zero_hbm, out_hbm,
                     zb0, zb1, ib0, ib1, tbuf, acc, zs0, zs1, is0, is1):
    c = lax.axis_index("c")
    s = lax.axis_index("s")
    w = c * NS + s
    row_base = w * RPW
    zbufs = (zb0, zb1)
    ibufs = (ib0, ib1)
    zsems = (zs0, zs1)
    isems = (is0, is1)

    zr0 = pl.multiple_of(s * SEG_PER_TILE, 8)
    pltpu.sync_copy(zero_hbm.at[pl.ds(zr0, SEG_PER_TILE)],
                    acc.at[pl.ds(zr0, SEG_PER_TILE)])

    @pl.when(s == NS - 1)
    def _zero_tail():
        t0 = pl.multiple_of(NS * SEG_PER_TILE, 8)
        pltpu.sync_copy(zero_hbm.at[pl.ds(t0, SEG_TAIL)],
                        acc.at[pl.ds(t0, SEG_TAIL)])

    plsc.subcore_barrier()

    def zsrc(l):
        return z_hbm.at[pl.ds(pl.multiple_of(row_base + l * K, 8), K)]

    def isrc(l):
        return b_hbm.at[pl.ds(pl.multiple_of(row_base + l * K, 8), K)]

    for b in range(NBUF):
        pltpu.async_copy(zsrc(b), zbufs[b], zsems[b])
        pltpu.async_copy(isrc(b), ibufs[b], isems[b])

    def body(i, carry):
        for b in range(NBUF):
            l = i * NBUF + b
            pltpu.make_async_copy(zsrc(l), zbufs[b], zsems[b]).wait()
            pltpu.make_async_copy(isrc(l), ibufs[b], isems[b]).wait()
            pltpu.sync_copy(zbufs[b], acc.at[ibufs[b]], add=True)
            nl = l + NBUF

            @pl.when(nl < NK)
            def _next():
                pltpu.async_copy(zsrc(nl), zbufs[b], zsems[b])
                pltpu.async_copy(isrc(nl), ibufs[b], isems[b])

        return carry

    lax.fori_loop(0, NK // NBUF, body, 0)

    t0 = pl.multiple_of(row_base + NK * K, 8)
    pltpu.sync_copy(z_hbm.at[pl.ds(t0, KTAIL)], zb0.at[pl.ds(0, KTAIL)])
    pltpu.sync_copy(b_hbm.at[pl.ds(t0, KTAIL)], tbuf)
    pltpu.sync_copy(zb0.at[pl.ds(0, KTAIL)], acc.at[tbuf], add=True)

    plsc.subcore_barrier()

    pltpu.sync_copy(acc.at[pl.ds(zr0, SEG_PER_TILE)],
                    out_hbm.at[c].at[pl.ds(zr0, SEG_PER_TILE)])

    @pl.when(s == NS - 1)
    def _write_tail():
        t0w = pl.multiple_of(NS * SEG_PER_TILE, 8)
        pltpu.sync_copy(acc.at[pl.ds(t0w, SEG_TAIL)],
                        out_hbm.at[c].at[pl.ds(t0w, SEG_TAIL)])


def _sc_scatter(z, bidx, zero):
    mesh = plsc.VectorSubcoreMesh(
        core_axis_name="c", subcore_axis_name="s", num_cores=NC, num_subcores=NS
    )
    return pl.kernel(
        _sc_scatter_body,
        out_type=jax.ShapeDtypeStruct((NC, S, D_IN), jnp.float32),
        mesh=mesh,
        scratch_types=[
            pltpu.VMEM((K, D_IN), jnp.float32),
            pltpu.VMEM((K, D_IN), jnp.float32),
            pltpu.VMEM((K,), jnp.int32),
            pltpu.VMEM((K,), jnp.int32),
            pltpu.VMEM((KTAIL,), jnp.int32),
            pltpu.VMEM_SHARED((S, D_IN), jnp.float32),
            pltpu.SemaphoreType.DMA,
            pltpu.SemaphoreType.DMA,
            pltpu.SemaphoreType.DMA,
            pltpu.SemaphoreType.DMA,
        ],
    )(z, bidx, zero)


def _combine_body(p_ref, b2_ref, o_ref):
    o_ref[...] = p_ref[0] + p_ref[1] + b2_ref[...]


def _combine(partials, b2):
    return pl.pallas_call(
        _combine_body,
        grid=(10,),
        in_specs=[
            pl.BlockSpec((NC, S // 10, D_IN), lambda i: (0, i, 0)),
            pl.BlockSpec((1, D_IN), lambda i: (0, 0)),
        ],
        out_specs=pl.BlockSpec((S // 10, D_IN), lambda i: (i, 0)),
        out_shape=jax.ShapeDtypeStruct((S, D_IN), jnp.float32),
    )(partials, b2.reshape(1, D_IN))


def kernel(x, batch, W1, b1, W2, b2):
    bidx = batch.astype(jnp.int32)
    z = _compute_z(x, W1.T.astype(jnp.bfloat16), b1, W2.T.astype(jnp.bfloat16))
    zero = jnp.zeros((S, D_IN), jnp.float32)
    partials = _sc_scatter(z, bidx, zero)
    return _combine(partials, b2)


# speedup vs baseline: 6.5659x; 1.0104x over previous
import functools

import jax
import jax.numpy as jnp
from jax import lax
from jax.experimental import pallas as pl
from jax.experimental.pallas import tpu as pltpu
from jax.experimental.pallas import tpu_sc as plsc

N = 320000
D_IN = 128
D_HID = 256
S = 10000

NSPLIT = 2
NPART = N // NSPLIT
RB = 16000
NC = 2
NS = 16
NW = NC * NS
RPW = NPART // NW
K = 128
NK = RPW // K
KTAIL = RPW - NK * K
NBUF = 2
SEG_PER_TILE = 624
SEG_TAIL = S - NS * SEG_PER_TILE

assert NPART % NW == 0 and KTAIL % 8 == 0 and NPART % RB == 0


def _z_body(x_ref, w1t_ref, b1_ref, w2t_ref, z_ref):
    xb = x_ref[...].astype(jnp.bfloat16)
    h = jnp.dot(xb, w1t_ref[...], preferred_element_type=jnp.float32)
    phi = jnp.tanh(h + b1_ref[...])
    z_ref[...] = jnp.dot(phi.astype(jnp.bfloat16), w2t_ref[...],
                         preferred_element_type=jnp.float32)


def _compute_z(x, W1T, b1, W2T, part):
    nb = NPART // RB
    return pl.pallas_call(
        _z_body,
        grid=(nb,),
        in_specs=[
            pl.BlockSpec((RB, D_IN), lambda i: (i + part * nb, 0)),
            pl.BlockSpec((D_IN, D_HID), lambda i: (0, 0)),
            pl.BlockSpec((1, D_HID), lambda i: (0, 0)),
            pl.BlockSpec((D_HID, D_IN), lambda i: (0, 0)),
        ],
        out_specs=pl.BlockSpec((RB, D_IN), lambda i: (i, 0)),
        out_shape=jax.ShapeDtypeStruct((NPART, D_IN), jnp.float32),
    )(x, W1T, b1.reshape(1, D_HID), W2T)


def _sc_scatter_body(part, z_hbm, b_hbm, zero_hbm, out_hbm,
                     zb0, zb1, ib0, ib1, tbuf, acc, zs0, zs1, is0, is1):
    c = lax.axis_index("c")
    s = lax.axis_index("s")
    w = c * NS + s
    row_base = w * RPW
    id_base = part * NPART + w * RPW
    zbufs = (zb0, zb1)
    ibufs = (ib0, ib1)
    zsems = (zs0, zs1)
    isems = (is0, is1)

    zr0 = pl.multiple_of(s * SEG_PER_TILE, 8)
    pltpu.sync_copy(zero_hbm.at[pl.ds(zr0, SEG_PER_TILE)],
                    acc.at[pl.ds(zr0, SEG_PER_TILE)])

    @pl.when(s == NS - 1)
    def _zero_tail():
        t0 = pl.multiple_of(NS * SEG_PER_TILE, 8)
        pltpu.sync_copy(zero_hbm.at[pl.ds(t0, SEG_TAIL)],
                        acc.at[pl.ds(t0, SEG_TAIL)])

    plsc.subcore_barrier()

    def zsrc(l):
        return z_hbm.at[pl.ds(pl.multiple_of(row_base + l * K, 8), K)]

    def isrc(l):
        return b_hbm.at[pl.ds(pl.multiple_of(id_base + l * K, 8), K)]

    def wait_and_scatter(l, b):
        pltpu.make_async_copy(zsrc(l), zbufs[b], zsems[b]).wait()
        pltpu.make_async_copy(isrc(l), ibufs[b], isems[b]).wait()
        pltpu.sync_copy(zbufs[b], acc.at[ibufs[b]], add=True)

    for b in range(NBUF):
        pltpu.async_copy(zsrc(b), zbufs[b], zsems[b])
        pltpu.async_copy(isrc(b), ibufs[b], isems[b])

    def body(i, carry):
        for b in range(NBUF):
            l = i * NBUF + b
            wait_and_scatter(l, b)
            nl = l + NBUF

            @pl.when(nl < NK)
            def _next():
                pltpu.async_copy(zsrc(nl), zbufs[b], zsems[b])
                pltpu.async_copy(isrc(nl), ibufs[b], isems[b])

        return carry

    lax.fori_loop(0, NK // NBUF, body, 0)
    for l in range(NK - NK % NBUF, NK):
        wait_and_scatter(l, l % NBUF)

    if KTAIL:
        t0 = pl.multiple_of(row_base + NK * K, 8)
        ti0 = pl.multiple_of(id_base + NK * K, 8)
        pltpu.sync_copy(z_hbm.at[pl.ds(t0, KTAIL)], zb0.at[pl.ds(0, KTAIL)])
        pltpu.sync_copy(b_hbm.at[pl.ds(ti0, KTAIL)], tbuf)
        pltpu.sync_copy(zb0.at[pl.ds(0, KTAIL)], acc.at[tbuf], add=True)

    plsc.subcore_barrier()

    pltpu.sync_copy(acc.at[pl.ds(zr0, SEG_PER_TILE)],
                    out_hbm.at[c].at[pl.ds(zr0, SEG_PER_TILE)])

    @pl.when(s == NS - 1)
    def _write_tail():
        t0w = pl.multiple_of(NS * SEG_PER_TILE, 8)
        pltpu.sync_copy(acc.at[pl.ds(t0w, SEG_TAIL)],
                        out_hbm.at[c].at[pl.ds(t0w, SEG_TAIL)])


def _sc_scatter(z, bidx, zero, part):
    mesh = plsc.VectorSubcoreMesh(
        core_axis_name="c", subcore_axis_name="s", num_cores=NC, num_subcores=NS
    )
    return pl.kernel(
        functools.partial(_sc_scatter_body, part),
        out_type=jax.ShapeDtypeStruct((NC, S, D_IN), jnp.float32),
        mesh=mesh,
        scratch_types=[
            pltpu.VMEM((K, D_IN), jnp.float32),
            pltpu.VMEM((K, D_IN), jnp.float32),
            pltpu.VMEM((K,), jnp.int32),
            pltpu.VMEM((K,), jnp.int32),
            pltpu.VMEM((max(KTAIL, 8),), jnp.int32),
            pltpu.VMEM_SHARED((S, D_IN), jnp.float32),
            pltpu.SemaphoreType.DMA,
            pltpu.SemaphoreType.DMA,
            pltpu.SemaphoreType.DMA,
            pltpu.SemaphoreType.DMA,
        ],
    )(z, bidx, zero)


def _combine_body(*refs):
    p_refs, b2_ref, o_ref = refs[:-2], refs[-2], refs[-1]
    total = b2_ref[...]
    for p in p_refs:
        total = total + p[0] + p[1]
    o_ref[...] = total


def _combine(partials, b2):
    return pl.pallas_call(
        _combine_body,
        grid=(10,),
        in_specs=[pl.BlockSpec((NC, S // 10, D_IN), lambda i: (0, i, 0))
                  for _ in partials]
        + [pl.BlockSpec((1, D_IN), lambda i: (0, 0))],
        out_specs=pl.BlockSpec((S // 10, D_IN), lambda i: (i, 0)),
        out_shape=jax.ShapeDtypeStruct((S, D_IN), jnp.float32),
    )(*partials, b2.reshape(1, D_IN))


def kernel(x, batch, W1, b1, W2, b2):
    bidx = batch.astype(jnp.int32)
    w1t = W1.T.astype(jnp.bfloat16)
    w2t = W2.T.astype(jnp.bfloat16)
    zero = jnp.zeros((S, D_IN), jnp.float32)
    partials = []
    for part in range(NSPLIT):
        z = _compute_z(x, w1t, b1, w2t, part)
        partials.append(_sc_scatter(z, bidx, zero, part))
    return _combine(partials, b2)
